# initial kernel scaffold (unmeasured)
import jax
import jax.numpy as jnp
from jax import lax
from jax.experimental import pallas as pl
from jax.experimental.pallas import tpu as pltpu

N_DEV = 4
EPS = 1e-5


def kernel(x, gamma, beta):
    m, n_per = x.shape
    assert m % 128 == 0
    r = m // 128

    gamma2 = gamma.reshape(1, n_per)
    beta2 = beta.reshape(1, n_per)

    def body(x_ref, g_ref, b_ref, out_ref, comm_ref, send_sems, recv_sems):
        my = lax.axis_index("i")

        barrier_sem = pltpu.get_barrier_semaphore()
        for k in range(1, N_DEV):
            peer = lax.rem(my + k, N_DEV)
            pl.semaphore_signal(
                barrier_sem, inc=1,
                device_id=(peer,), device_id_type=pl.DeviceIdType.MESH,
            )
        pl.semaphore_wait(barrier_sem, N_DEV - 1)

        xf = x_ref[:, :].astype(jnp.float32).reshape(r, 128, n_per)
        s1 = jnp.sum(xf, axis=2)
        s2 = jnp.sum(xf * xf, axis=2)
        comm_ref[my] = jnp.stack([s1, s2], axis=0)

        sends = []
        for k in range(1, N_DEV):
            peer = lax.rem(my + k, N_DEV)
            rdma = pltpu.make_async_remote_copy(
                src_ref=comm_ref.at[my],
                dst_ref=comm_ref.at[my],
                send_sem=send_sems.at[k - 1],
                recv_sem=recv_sems.at[my],
                device_id=(peer,),
                device_id_type=pl.DeviceIdType.MESH,
            )
            rdma.start()
            sends.append(rdma)

        for k in range(1, N_DEV):
            src = lax.rem(my + k, N_DEV)
            recv = pltpu.make_async_remote_copy(
                src_ref=comm_ref.at[src],
                dst_ref=comm_ref.at[src],
                send_sem=send_sems.at[k - 1],
                recv_sem=recv_sems.at[src],
                device_id=(src,),
                device_id_type=pl.DeviceIdType.MESH,
            )
            recv.wait_recv()

        total = comm_ref[0] + comm_ref[1] + comm_ref[2] + comm_ref[3]
        mean = total[0] * (1.0 / (N_DEV * n_per))
        ex2 = total[1] * (1.0 / (N_DEV * n_per))
        inv = lax.rsqrt(ex2 - mean * mean + EPS)

        g = g_ref[:, :].astype(jnp.float32).reshape(1, 1, n_per)
        b = b_ref[:, :].astype(jnp.float32).reshape(1, 1, n_per)
        out3 = (xf - mean[:, :, None]) * inv[:, :, None] * g + b
        out_ref[:, :] = out3.reshape(m, n_per).astype(out_ref.dtype)

        for rdma in sends:
            rdma.wait_send()

    out_shape = jax.ShapeDtypeStruct((m, n_per), jnp.bfloat16)
    return pl.pallas_call(
        body,
        out_shape=out_shape,
        in_specs=[
            pl.BlockSpec(memory_space=pltpu.VMEM),
            pl.BlockSpec(memory_space=pltpu.VMEM),
            pl.BlockSpec(memory_space=pltpu.VMEM),
        ],
        out_specs=pl.BlockSpec(memory_space=pltpu.VMEM),
        scratch_shapes=[
            pltpu.VMEM((N_DEV, 2, r, 128), jnp.float32),
            pltpu.SemaphoreType.DMA((N_DEV - 1,)),
            pltpu.SemaphoreType.DMA((N_DEV,)),
        ],
        compiler_params=pltpu.CompilerParams(collective_id=0),
    )(x, gamma2, beta2)


# baseline (device time: 28364 ns/iter reference)
import jax
import jax.numpy as jnp
from jax import lax
from jax.experimental import pallas as pl
from jax.experimental.pallas import tpu as pltpu

N_DEV = 4
EPS = 1e-5


def kernel(x, gamma, beta):
    m, n_per = x.shape
    assert m % 128 == 0
    r = m // 128

    gamma2 = gamma.reshape(1, n_per)
    beta2 = beta.reshape(1, n_per)

    def body(x_ref, g_ref, b_ref, out_ref, comm_ref, send_sems, recv_sems):
        my = lax.axis_index("i")

        barrier_sem = pltpu.get_barrier_semaphore()
        for k in range(1, N_DEV):
            peer = lax.rem(my + k, N_DEV)
            pl.semaphore_signal(
                barrier_sem, inc=1,
                device_id=(peer,), device_id_type=pl.DeviceIdType.MESH,
            )
        pl.semaphore_wait(barrier_sem, N_DEV - 1)

        xf = x_ref[:, :].astype(jnp.float32).reshape(r, 128, n_per)
        s1 = jnp.sum(xf, axis=2)
        s2 = jnp.sum(xf * xf, axis=2)
        comm_ref[my] = jnp.stack([s1, s2], axis=0)

        sends = []
        for k in range(1, N_DEV):
            peer = lax.rem(my + k, N_DEV)
            rdma = pltpu.make_async_remote_copy(
                src_ref=comm_ref.at[my],
                dst_ref=comm_ref.at[my],
                send_sem=send_sems.at[k - 1],
                recv_sem=recv_sems.at[my],
                device_id=(peer,),
                device_id_type=pl.DeviceIdType.MESH,
            )
            rdma.start()
            sends.append(rdma)

        for k in range(1, N_DEV):
            src = lax.rem(my + k, N_DEV)
            recv = pltpu.make_async_remote_copy(
                src_ref=comm_ref.at[src],
                dst_ref=comm_ref.at[src],
                send_sem=send_sems.at[k - 1],
                recv_sem=recv_sems.at[src],
                device_id=(src,),
                device_id_type=pl.DeviceIdType.MESH,
            )
            recv.wait_recv()

        total = comm_ref[0] + comm_ref[1] + comm_ref[2] + comm_ref[3]
        mean = total[0] * (1.0 / (N_DEV * n_per))
        ex2 = total[1] * (1.0 / (N_DEV * n_per))
        inv = lax.rsqrt(ex2 - mean * mean + EPS)

        g = g_ref[:, :].astype(jnp.float32).reshape(1, 1, n_per)
        b = b_ref[:, :].astype(jnp.float32).reshape(1, 1, n_per)
        out3 = (xf - mean[:, :, None]) * inv[:, :, None] * g + b
        out_ref[:, :] = out3.reshape(m, n_per).astype(out_ref.dtype)

        for rdma in sends:
            rdma.wait_send()

    out_shape = jax.ShapeDtypeStruct((m, n_per), jnp.bfloat16)
    return pl.pallas_call(
        body,
        out_shape=out_shape,
        in_specs=[
            pl.BlockSpec(memory_space=pltpu.VMEM),
            pl.BlockSpec(memory_space=pltpu.VMEM),
            pl.BlockSpec(memory_space=pltpu.VMEM),
        ],
        out_specs=pl.BlockSpec(memory_space=pltpu.VMEM),
        scratch_shapes=[
            pltpu.VMEM((N_DEV, 2, r, 128), jnp.float32),
            pltpu.SemaphoreType.DMA((N_DEV - 1,)),
            pltpu.SemaphoreType.DMA((N_DEV,)),
        ],
        compiler_params=pltpu.CompilerParams(
            collective_id=0,
            vmem_limit_bytes=100 * 1024 * 1024,
        ),
    )(x, gamma2, beta2)


# device time: 26581 ns/iter; 1.0671x vs baseline; 1.0671x over previous
import jax
import jax.numpy as jnp
from jax import lax
from jax.experimental import pallas as pl
from jax.experimental.pallas import tpu as pltpu

N_DEV = 4
EPS = 1e-5
C = 2


def kernel(x, gamma, beta):
    m, n_per = x.shape
    r = m // 128
    rc = r // C
    mc = m // C
    assert m % (128 * C) == 0

    gamma2 = gamma.reshape(1, n_per)
    beta2 = beta.reshape(1, n_per)

    def body(x_ref, g_ref, b_ref, out_ref, comm_ref, send_sems, recv_sems):
        my = lax.axis_index("i")

        barrier_sem = pltpu.get_barrier_semaphore()
        for k in range(1, N_DEV):
            peer = lax.rem(my + k, N_DEV)
            pl.semaphore_signal(
                barrier_sem, inc=1,
                device_id=(peer,), device_id_type=pl.DeviceIdType.MESH,
            )
        pl.semaphore_wait(barrier_sem, N_DEV - 1)

        sends = []
        for c in range(C):
            xf = (
                x_ref[pl.ds(c * mc, mc), :]
                .astype(jnp.float32)
                .reshape(rc, 128, n_per)
            )
            s1 = jnp.sum(xf, axis=2)
            s2 = jnp.sum(xf * xf, axis=2)
            comm_ref[c, my] = jnp.stack([s1, s2], axis=0)

            for k in range(1, N_DEV):
                peer = lax.rem(my + k, N_DEV)
                rdma = pltpu.make_async_remote_copy(
                    src_ref=comm_ref.at[c, my],
                    dst_ref=comm_ref.at[c, my],
                    send_sem=send_sems.at[c, k - 1],
                    recv_sem=recv_sems.at[c, my],
                    device_id=(peer,),
                    device_id_type=pl.DeviceIdType.MESH,
                )
                rdma.start()
                sends.append(rdma)

        for c in range(C):
            for k in range(1, N_DEV):
                src = lax.rem(my + k, N_DEV)
                recv = pltpu.make_async_remote_copy(
                    src_ref=comm_ref.at[c, src],
                    dst_ref=comm_ref.at[c, src],
                    send_sem=send_sems.at[c, k - 1],
                    recv_sem=recv_sems.at[c, src],
                    device_id=(src,),
                    device_id_type=pl.DeviceIdType.MESH,
                )
                recv.wait_recv()

            total = (
                comm_ref[c, 0] + comm_ref[c, 1] + comm_ref[c, 2] + comm_ref[c, 3]
            )
            mean = total[0] * (1.0 / (N_DEV * n_per))
            ex2 = total[1] * (1.0 / (N_DEV * n_per))
            inv = lax.rsqrt(ex2 - mean * mean + EPS)

            g = g_ref[:, :].astype(jnp.float32).reshape(1, 1, n_per)
            b = b_ref[:, :].astype(jnp.float32).reshape(1, 1, n_per)
            xf = (
                x_ref[pl.ds(c * mc, mc), :]
                .astype(jnp.float32)
                .reshape(rc, 128, n_per)
            )
            out3 = (xf - mean[:, :, None]) * inv[:, :, None] * g + b
            out_ref[pl.ds(c * mc, mc), :] = out3.reshape(mc, n_per).astype(
                out_ref.dtype
            )

        for rdma in sends:
            rdma.wait_send()

    out_shape = jax.ShapeDtypeStruct((m, n_per), jnp.bfloat16)
    return pl.pallas_call(
        body,
        out_shape=out_shape,
        in_specs=[
            pl.BlockSpec(memory_space=pltpu.VMEM),
            pl.BlockSpec(memory_space=pltpu.VMEM),
            pl.BlockSpec(memory_space=pltpu.VMEM),
        ],
        out_specs=pl.BlockSpec(memory_space=pltpu.VMEM),
        scratch_shapes=[
            pltpu.VMEM((C, N_DEV, 2, rc, 128), jnp.float32),
            pltpu.SemaphoreType.DMA((C, N_DEV - 1)),
            pltpu.SemaphoreType.DMA((C, N_DEV)),
        ],
        compiler_params=pltpu.CompilerParams(
            collective_id=0,
            vmem_limit_bytes=100 * 1024 * 1024,
        ),
    )(x, gamma2, beta2)


# device time: 25705 ns/iter; 1.1034x vs baseline; 1.0341x over previous
import jax
import jax.numpy as jnp
from jax import lax
from jax.experimental import pallas as pl
from jax.experimental.pallas import tpu as pltpu

N_DEV = 4
EPS = 1e-5
C = 2


def kernel(x, gamma, beta):
    m, n_per = x.shape
    r = m // 128
    rc = r // C
    mc = m // C
    assert m % (128 * C) == 0

    gamma2 = gamma.reshape(1, n_per)
    beta2 = beta.reshape(1, n_per)

    def body(x_ref, g_ref, b_ref, out_ref, comm_ref, send_sems, recv_sems):
        my = lax.axis_index("i")

        barrier_sem = pltpu.get_barrier_semaphore()
        for k in range(1, N_DEV):
            peer = lax.rem(my + k, N_DEV)
            pl.semaphore_signal(
                barrier_sem, inc=1,
                device_id=(peer,), device_id_type=pl.DeviceIdType.MESH,
            )
        pl.semaphore_wait(barrier_sem, N_DEV - 1)

        sends = []
        for c in range(C):
            xf = (
                x_ref[pl.ds(c * mc, mc), :]
                .astype(jnp.float32)
                .reshape(rc, 128, n_per)
            )
            s1 = jnp.sum(xf, axis=2)
            s2 = jnp.sum(xf * xf, axis=2)
            comm_ref[c, my] = jnp.stack([s1, s2], axis=0)

            for k in range(1, N_DEV):
                peer = lax.rem(my + k, N_DEV)
                rdma = pltpu.make_async_remote_copy(
                    src_ref=comm_ref.at[c, my],
                    dst_ref=comm_ref.at[c, my],
                    send_sem=send_sems.at[c, k - 1],
                    recv_sem=recv_sems.at[c, my],
                    device_id=(peer,),
                    device_id_type=pl.DeviceIdType.MESH,
                )
                rdma.start()
                sends.append(rdma)

        for c in range(C):
            for k in range(1, N_DEV):
                src = lax.rem(my + k, N_DEV)
                recv = pltpu.make_async_remote_copy(
                    src_ref=comm_ref.at[c, src],
                    dst_ref=comm_ref.at[c, src],
                    send_sem=send_sems.at[c, k - 1],
                    recv_sem=recv_sems.at[c, src],
                    device_id=(src,),
                    device_id_type=pl.DeviceIdType.MESH,
                )
                recv.wait_recv()

            total = (
                comm_ref[c, 0] + comm_ref[c, 1] + comm_ref[c, 2] + comm_ref[c, 3]
            )
            mean = total[0] * (1.0 / (N_DEV * n_per))
            ex2 = total[1] * (1.0 / (N_DEV * n_per))
            inv = lax.rsqrt(ex2 - mean * mean + EPS)

            g = g_ref[:, :].astype(jnp.bfloat16).reshape(1, 1, n_per)
            b = b_ref[:, :].astype(jnp.bfloat16).reshape(1, 1, n_per)
            mean_b = mean.astype(jnp.bfloat16)[:, :, None]
            inv_b = inv.astype(jnp.bfloat16)[:, :, None]
            xb = (
                x_ref[pl.ds(c * mc, mc), :]
                .astype(jnp.bfloat16)
                .reshape(rc, 128, n_per)
            )
            out3 = (xb - mean_b) * inv_b * g + b
            out_ref[pl.ds(c * mc, mc), :] = out3.reshape(mc, n_per)

        for rdma in sends:
            rdma.wait_send()

    out_shape = jax.ShapeDtypeStruct((m, n_per), jnp.bfloat16)
    return pl.pallas_call(
        body,
        out_shape=out_shape,
        in_specs=[
            pl.BlockSpec(memory_space=pltpu.VMEM),
            pl.BlockSpec(memory_space=pltpu.VMEM),
            pl.BlockSpec(memory_space=pltpu.VMEM),
        ],
        out_specs=pl.BlockSpec(memory_space=pltpu.VMEM),
        scratch_shapes=[
            pltpu.VMEM((C, N_DEV, 2, rc, 128), jnp.float32),
            pltpu.SemaphoreType.DMA((C, N_DEV - 1)),
            pltpu.SemaphoreType.DMA((C, N_DEV)),
        ],
        compiler_params=pltpu.CompilerParams(
            collective_id=0,
            vmem_limit_bytes=100 * 1024 * 1024,
        ),
    )(x, gamma2, beta2)


# device time: 21586 ns/iter; 1.3140x vs baseline; 1.1908x over previous
import jax
import jax.numpy as jnp
from jax import lax
from jax.experimental import pallas as pl
from jax.experimental.pallas import tpu as pltpu

N_DEV = 4
EPS = 1e-5
NC = 4


def kernel(x, gamma, beta):
    m, n_per = x.shape
    r = m // 128
    rc = r // NC
    mc = m // NC
    assert m % (128 * NC) == 0

    gamma2 = gamma.reshape(1, n_per)
    beta2 = beta.reshape(1, n_per)

    def body(
        x_ref, g_ref, b_ref, out_ref,
        xbuf, stage, comm_ref, in_sems, out_sems, send_sems, recv_sems,
    ):
        my = lax.axis_index("i")

        in_copies = []
        for c in range(NC):
            cp = pltpu.make_async_copy(
                x_ref.at[pl.ds(c * mc, mc), :],
                xbuf.at[pl.ds(c * mc, mc), :],
                in_sems.at[c],
            )
            cp.start()
            in_copies.append(cp)

        barrier_sem = pltpu.get_barrier_semaphore()
        for k in range(1, N_DEV):
            peer = lax.rem(my + k, N_DEV)
            pl.semaphore_signal(
                barrier_sem, inc=1,
                device_id=(peer,), device_id_type=pl.DeviceIdType.MESH,
            )
        pl.semaphore_wait(barrier_sem, N_DEV - 1)

        sends = []
        for c in range(NC):
            in_copies[c].wait()
            xf = xbuf[pl.ds(c * mc, mc), :].reshape(rc, 128, n_per)
            s1 = jnp.sum(xf, axis=2)
            s2 = jnp.sum(xf * xf, axis=2)
            comm_ref[c, my] = jnp.stack([s1, s2], axis=0)

            for k in range(1, N_DEV):
                peer = lax.rem(my + k, N_DEV)
                rdma = pltpu.make_async_remote_copy(
                    src_ref=comm_ref.at[c, my],
                    dst_ref=comm_ref.at[c, my],
                    send_sem=send_sems.at[c, k - 1],
                    recv_sem=recv_sems.at[c, my],
                    device_id=(peer,),
                    device_id_type=pl.DeviceIdType.MESH,
                )
                rdma.start()
                sends.append(rdma)

        g = g_ref[:, :].astype(jnp.bfloat16).reshape(1, 1, n_per)
        b = b_ref[:, :].astype(jnp.bfloat16).reshape(1, 1, n_per)
        out_copies = [None, None]
        for c in range(NC):
            for k in range(1, N_DEV):
                src = lax.rem(my + k, N_DEV)
                recv = pltpu.make_async_remote_copy(
                    src_ref=comm_ref.at[c, src],
                    dst_ref=comm_ref.at[c, src],
                    send_sem=send_sems.at[c, k - 1],
                    recv_sem=recv_sems.at[c, src],
                    device_id=(src,),
                    device_id_type=pl.DeviceIdType.MESH,
                )
                recv.wait_recv()

            total = (
                comm_ref[c, 0] + comm_ref[c, 1] + comm_ref[c, 2] + comm_ref[c, 3]
            )
            mean = total[0] * (1.0 / (N_DEV * n_per))
            ex2 = total[1] * (1.0 / (N_DEV * n_per))
            inv = lax.rsqrt(ex2 - mean * mean + EPS)

            mean_b = mean.astype(jnp.bfloat16)[:, :, None]
            inv_b = inv.astype(jnp.bfloat16)[:, :, None]
            xb = (
                xbuf[pl.ds(c * mc, mc), :]
                .astype(jnp.bfloat16)
                .reshape(rc, 128, n_per)
            )
            out3 = (xb - mean_b) * inv_b * g + b

            slot = c % 2
            if out_copies[slot] is not None:
                out_copies[slot].wait()
            stage[slot] = out3.reshape(mc, n_per)
            cp = pltpu.make_async_copy(
                stage.at[slot],
                out_ref.at[pl.ds(c * mc, mc), :],
                out_sems.at[slot],
            )
            cp.start()
            out_copies[slot] = cp

        for cp in out_copies:
            if cp is not None:
                cp.wait()
        for rdma in sends:
            rdma.wait_send()

    out_shape = jax.ShapeDtypeStruct((m, n_per), jnp.bfloat16)
    return pl.pallas_call(
        body,
        out_shape=out_shape,
        in_specs=[
            pl.BlockSpec(memory_space=pl.ANY),
            pl.BlockSpec(memory_space=pltpu.VMEM),
            pl.BlockSpec(memory_space=pltpu.VMEM),
        ],
        out_specs=pl.BlockSpec(memory_space=pl.ANY),
        scratch_shapes=[
            pltpu.VMEM((m, n_per), jnp.float32),
            pltpu.VMEM((2, mc, n_per), jnp.bfloat16),
            pltpu.VMEM((NC, N_DEV, 2, rc, 128), jnp.float32),
            pltpu.SemaphoreType.DMA((NC,)),
            pltpu.SemaphoreType.DMA((2,)),
            pltpu.SemaphoreType.DMA((NC, N_DEV - 1)),
            pltpu.SemaphoreType.DMA((NC, N_DEV)),
        ],
        compiler_params=pltpu.CompilerParams(
            collective_id=0,
            vmem_limit_bytes=100 * 1024 * 1024,
        ),
    )(x, gamma2, beta2)
